# Initial kernel scaffold; baseline (speedup 1.0000x reference)
import jax
import jax.numpy as jnp
from jax.experimental import pallas as pl


def kernel(x_raw, adj, dist_km):
    raise NotImplementedError("write your pallas kernel here")


# speedup vs baseline: 10.6203x; 10.6203x over previous
import functools

import jax
import jax.numpy as jnp
from jax.experimental import pallas as pl


def _transport_delay_kernel(x_ref, adj_ref, dist_ref, out_ref,
                            *, T, W, t_base):
    wspm_mean = 2.5
    wspm_scale = 1.8
    max_delay_hours = 24.0
    wind_w = 4
    wind_speed_idx = 10

    xs = x_ref[0]
    adj = adj_ref[0]
    dist = dist_ref[...]
    N = adj.shape[0]

    wind = xs[W - wind_w:, :, wind_speed_idx]
    wspm_raw = jnp.clip(jnp.mean(wind, axis=0) * wspm_scale + wspm_mean,
                        0.0, None)
    speed_kmh = wspm_raw * 3.6 + 0.001

    tau = jnp.clip(dist / speed_kmh[None, :], 0.0, max_delay_hours)
    t_query = float(T - 1) - tau
    t0f = jnp.floor(t_query)
    w1 = t_query - t0f
    r0 = t0f.astype(jnp.int32) - t_base
    r1 = jnp.minimum(r0 + 1, W - 1)

    tt = jax.lax.broadcasted_iota(jnp.int32, (N, W, N), 1)
    w1b = w1[:, None, :]
    C = adj[:, None, :] * (
        (1.0 - w1b) * (tt == r0[:, None, :]).astype(jnp.float32)
        + w1b * (tt == r1[:, None, :]).astype(jnp.float32))

    lhs = C.reshape(N, W * N)
    rhs = xs.reshape(W * N, xs.shape[-1])
    out_ref[0] = jnp.dot(lhs, rhs, preferred_element_type=jnp.float32)


def kernel(x_raw, adj, dist_km):
    B, T, N, F = x_raw.shape
    W = 28
    t_base = T - W

    grid = (B,)
    return pl.pallas_call(
        functools.partial(_transport_delay_kernel, T=T, W=W, t_base=t_base),
        grid=grid,
        in_specs=[
            pl.BlockSpec((1, W, N, F), lambda b: (b, T // W - 1, 0, 0)),
            pl.BlockSpec((1, N, N), lambda b: (b, 0, 0)),
            pl.BlockSpec((N, N), lambda b: (0, 0)),
        ],
        out_specs=pl.BlockSpec((1, N, F), lambda b: (b, 0, 0)),
        out_shape=jax.ShapeDtypeStruct((B, N, F), jnp.float32),
    )(x_raw, adj, dist_km)


# speedup vs baseline: 25.7008x; 2.4200x over previous
import functools

import jax
import jax.numpy as jnp
from jax.experimental import pallas as pl


def _transport_delay_kernel(x_ref, adj_ref, dist_ref, out_ref,
                            *, T, W, t_base):
    wspm_mean = 2.5
    wspm_scale = 1.8
    max_delay_hours = 24.0
    wind_w = 4
    wind_speed_idx = 10

    xs = x_ref[0]
    adj = adj_ref[0]
    dist = dist_ref[...]
    N = adj.shape[0]

    wind = xs[W - wind_w:, :, wind_speed_idx]
    wspm_raw = jnp.clip(jnp.mean(wind, axis=0) * wspm_scale + wspm_mean,
                        0.0, None)
    speed_kmh = wspm_raw * 3.6 + 0.001

    tau = jnp.clip(dist / speed_kmh[None, :], 0.0, max_delay_hours)
    t_query = float(T - 1) - tau
    t0f = jnp.floor(t_query)
    w1 = t_query - t0f
    r0 = t0f.astype(jnp.int32) - t_base
    r1 = jnp.minimum(r0 + 1, W - 1)

    tt = jax.lax.broadcasted_iota(jnp.int32, (N, W, N), 1)
    w1b = w1[:, None, :]
    C = adj[:, None, :] * (
        (1.0 - w1b) * (tt == r0[:, None, :]).astype(jnp.float32)
        + w1b * (tt == r1[:, None, :]).astype(jnp.float32))

    lhs = C.reshape(N, W * N)
    rhs = xs.reshape(W * N, xs.shape[-1])
    out_ref[0] = jnp.dot(lhs, rhs, preferred_element_type=jnp.float32)


def kernel(x_raw, adj, dist_km):
    B, T, N, F = x_raw.shape
    W = 28
    t_base = T - W
    xs = jax.lax.slice(x_raw, (0, t_base, 0, 0), (B, T, N, F))

    grid = (B,)
    return pl.pallas_call(
        functools.partial(_transport_delay_kernel, T=T, W=W, t_base=t_base),
        grid=grid,
        in_specs=[
            pl.BlockSpec((1, W, N, F), lambda b: (b, 0, 0, 0)),
            pl.BlockSpec((1, N, N), lambda b: (b, 0, 0)),
            pl.BlockSpec((N, N), lambda b: (0, 0)),
        ],
        out_specs=pl.BlockSpec((1, N, F), lambda b: (b, 0, 0)),
        out_shape=jax.ShapeDtypeStruct((B, N, F), jnp.float32),
    )(xs, adj, dist_km)


# speedup vs baseline: 32.4921x; 1.2642x over previous
import functools

import jax
import jax.numpy as jnp
from jax.experimental import pallas as pl
from jax.experimental.pallas import tpu as pltpu


def _transport_delay_kernel(x_ref, adj_ref, dist_ref, out_ref, lhs_ref,
                            *, T, W, t_base):
    wspm_mean = 2.5
    wspm_scale = 1.8
    max_delay_hours = 24.0
    wind_w = 4
    wind_speed_idx = 10

    xs = x_ref[0]
    adj = adj_ref[0]
    dist = dist_ref[...]
    N = adj.shape[0]

    wind = xs[W - wind_w:, :, wind_speed_idx]
    wspm_raw = jnp.clip(jnp.mean(wind, axis=0) * wspm_scale + wspm_mean,
                        0.0, None)
    speed_kmh = wspm_raw * 3.6 + 0.001

    tau = jnp.clip(dist / speed_kmh[None, :], 0.0, max_delay_hours)
    t_query = float(T - 1) - tau

    aw = adj
    for t in range(W):
        t_abs = float(t_base + t)
        w = jnp.maximum(1.0 - jnp.abs(t_query - t_abs), 0.0)
        lhs_ref[:, t * N:(t + 1) * N] = aw * w

    rhs = xs.reshape(W * N, xs.shape[-1])
    out_ref[0] = jnp.dot(lhs_ref[...], rhs, preferred_element_type=jnp.float32)


def kernel(x_raw, adj, dist_km):
    B, T, N, F = x_raw.shape
    W = 25
    t_base = T - W
    xs = jax.lax.slice(x_raw, (0, t_base, 0, 0), (B, T, N, F))

    grid = (B,)
    return pl.pallas_call(
        functools.partial(_transport_delay_kernel, T=T, W=W, t_base=t_base),
        grid=grid,
        in_specs=[
            pl.BlockSpec((1, W, N, F), lambda b: (b, 0, 0, 0)),
            pl.BlockSpec((1, N, N), lambda b: (b, 0, 0)),
            pl.BlockSpec((N, N), lambda b: (0, 0)),
        ],
        out_specs=pl.BlockSpec((1, N, F), lambda b: (b, 0, 0)),
        out_shape=jax.ShapeDtypeStruct((B, N, F), jnp.float32),
        scratch_shapes=[pltpu.VMEM((N, W * N), jnp.float32)],
    )(xs, adj, dist_km)
